# Initial kernel scaffold; baseline (speedup 1.0000x reference)
import jax
import jax.numpy as jnp
from jax.experimental import pallas as pl


def kernel(x, edge_index):
    raise NotImplementedError("write your pallas kernel here")


# speedup vs baseline: 12.0859x; 12.0859x over previous
import functools

import jax
import jax.numpy as jnp
from jax import lax
from jax.experimental import pallas as pl
from jax.experimental.pallas import tpu as pltpu
from jax.experimental.pallas import tpu_sc as plsc

N_NODES = 10000
N_EDGES = 320000
D_FEAT = 128
DH = 64
NPAD = 10240
RPT = 640
CR = 80
NRC = RPT // CR
EC = 128
MAIN_CH = 156
MAX_CH = 157
TAIL0 = MAIN_CH * EC * 16


def _rsqrt16(d):
    i = lax.bitcast_convert_type(d, jnp.int32)
    i = jnp.int32(0x5F3759DF) - lax.shift_right_arithmetic(i, 1)
    r = lax.bitcast_convert_type(i, jnp.float32)
    for _ in range(3):
        r = r * (jnp.float32(1.5) - jnp.float32(0.5) * d * r * r)
    return jnp.where(d > 0, r, jnp.float32(0.0))


def _body(x_hbm, src_hbm, dst_hbm, out_hbm, y_hbm,
          acc_sh, deg_sh,
          src_v, dst_v, rowbuf, tbuf, degbuf, ones_v, zeros_v):
    c = lax.axis_index("c")
    t = lax.axis_index("s")
    col0 = c * DH
    ebase = t * (MAIN_CH * EC)
    has_tail = t < 4
    nch = jnp.where(has_tail, MAX_CH, MAIN_CH)
    row_t = t * RPT

    def _fill_ones(i, _):
        ones_v[i, pl.ds(0, 16)] = jnp.full((16,), 1.0, jnp.float32)
        return 0
    lax.fori_loop(0, EC, _fill_ones, 0)

    def _fill_zeros(r, _):
        for cc in range(DH // 16):
            zeros_v[r, pl.ds(cc * 16, 16)] = jnp.zeros((16,), jnp.float32)
        return 0
    lax.fori_loop(0, CR, _fill_zeros, 0)

    pltpu.sync_copy(src_hbm.at[pl.ds(ebase, MAIN_CH * EC)],
                    src_v.at[pl.ds(0, MAIN_CH * EC)])

    def _ld_dst(j, _):
        pltpu.sync_copy(dst_hbm.at[pl.ds(ebase + j * EC, EC)], dst_v.at[j])
        return 0
    lax.fori_loop(0, MAIN_CH, _ld_dst, 0)

    @pl.when(has_tail)
    def _():
        pltpu.sync_copy(src_hbm.at[pl.ds(TAIL0 + t * EC, EC)],
                        src_v.at[pl.ds(MAIN_CH * EC, EC)])
        pltpu.sync_copy(dst_hbm.at[pl.ds(TAIL0 + t * EC, EC)],
                        dst_v.at[MAIN_CH])

    def _zdeg(k, _):
        pltpu.sync_copy(zeros_v.at[pl.ds(0, CR), pl.ds(0, 16)],
                        deg_sh.at[pl.ds(row_t + k * CR, CR), :])
        return 0
    lax.fori_loop(0, NRC, _zdeg, 0)
    for k in range(NRC):
        pltpu.sync_copy(zeros_v, acc_sh.at[pl.ds(row_t + k * CR, CR), :])
    plsc.subcore_barrier()

    def _deg_ch(j, _):
        pltpu.sync_copy(ones_v, deg_sh.at[dst_v.at[j]], add=True)
        return 0
    lax.fori_loop(0, nch, _deg_ch, 0)
    plsc.subcore_barrier()

    pltpu.sync_copy(deg_sh.at[pl.ds(row_t, RPT), :], degbuf)

    def _dis_row(r, _):
        v = degbuf[r, pl.ds(0, 16)]
        degbuf[r, pl.ds(0, 16)] = _rsqrt16(v)
        return 0
    lax.fori_loop(0, RPT, _dis_row, 0)

    for k in range(NRC):
        row0 = row_t + k * CR

        @pl.when(row0 < N_NODES)
        def _():
            pltpu.sync_copy(x_hbm.at[pl.ds(row0, CR), pl.ds(col0, DH)], tbuf)

            def _scale_row(r, _):
                d = degbuf[k * CR + r, pl.ds(0, 16)]
                for cc in range(DH // 16):
                    sl = pl.ds(cc * 16, 16)
                    tbuf[r, sl] = tbuf[r, sl] * d
                return 0
            lax.fori_loop(0, CR, _scale_row, 0)
            pltpu.sync_copy(tbuf, y_hbm.at[c, pl.ds(row0, CR), :])
    plsc.subcore_barrier()

    for layer in range(3):
        last = layer == 2

        def _edge_ch(j, _):
            pltpu.sync_copy(y_hbm.at[c].at[src_v.at[pl.ds(j * EC, EC)]],
                            rowbuf)
            pltpu.sync_copy(rowbuf, acc_sh.at[dst_v.at[j]], add=True)
            return 0
        lax.fori_loop(0, nch, _edge_ch, 0)
        plsc.subcore_barrier()

        for k in range(NRC):
            row0 = row_t + k * CR

            @pl.when(row0 < N_NODES)
            def _():
                pltpu.sync_copy(acc_sh.at[pl.ds(row0, CR), :], tbuf)

                def _trow(r, _):
                    d = degbuf[k * CR + r, pl.ds(0, 16)]
                    for cc in range(DH // 16):
                        sl = pl.ds(cc * 16, 16)
                        v = tbuf[r, sl]
                        v = jnp.maximum(v * d, jnp.float32(0.0))
                        if not last:
                            v = v * d
                        tbuf[r, sl] = v
                    return 0
                lax.fori_loop(0, CR, _trow, 0)
                if last:
                    pltpu.sync_copy(
                        tbuf, out_hbm.at[pl.ds(row0, CR), pl.ds(col0, DH)])
                else:
                    pltpu.sync_copy(tbuf, y_hbm.at[c, pl.ds(row0, CR), :])
                    pltpu.sync_copy(zeros_v,
                                    acc_sh.at[pl.ds(row0, CR), :])
        if not last:
            plsc.subcore_barrier()


@jax.jit
def _run(x, src, dst):
    mesh = plsc.VectorSubcoreMesh(core_axis_name="c", subcore_axis_name="s")
    f = pl.kernel(
        _body,
        out_type=(
            jax.ShapeDtypeStruct((N_NODES, D_FEAT), jnp.float32),
            jax.ShapeDtypeStruct((2, NPAD, DH), jnp.float32),
        ),
        mesh=mesh,
        compiler_params=pltpu.CompilerParams(use_tc_tiling_on_sc=False),
        scratch_types=[
            pltpu.VMEM_SHARED((NPAD, DH), jnp.float32),
            pltpu.VMEM_SHARED((NPAD, 16), jnp.float32),
            pltpu.VMEM((MAX_CH * EC,), jnp.int32),
            pltpu.VMEM((MAX_CH, EC), jnp.int32),
            pltpu.VMEM((EC, DH), jnp.float32),
            pltpu.VMEM((CR, DH), jnp.float32),
            pltpu.VMEM((RPT, 16), jnp.float32),
            pltpu.VMEM((EC, 16), jnp.float32),
            pltpu.VMEM((CR, DH), jnp.float32),
        ],
    )
    out, _ = f(x, src, dst)
    return out


def kernel(x, edge_index):
    ei = edge_index.astype(jnp.int32)
    return _run(x, ei[0], ei[1])


# speedup vs baseline: 17.1536x; 1.4193x over previous
import functools

import jax
import jax.numpy as jnp
from jax import lax
from jax.experimental import pallas as pl
from jax.experimental.pallas import tpu as pltpu
from jax.experimental.pallas import tpu_sc as plsc

N_NODES = 10000
N_EDGES = 320000
D_FEAT = 128
DH = 64
NPAD = 10240
RPT = 640
CR = 80
NRC = RPT // CR
EC = 128
MAIN_CH = 156
MAX_CH = 157
TAIL0 = MAIN_CH * EC * 16


def _rsqrt16(d):
    i = lax.bitcast_convert_type(d, jnp.int32)
    i = jnp.int32(0x5F3759DF) - lax.shift_right_arithmetic(i, 1)
    r = lax.bitcast_convert_type(i, jnp.float32)
    for _ in range(3):
        r = r * (jnp.float32(1.5) - jnp.float32(0.5) * d * r * r)
    return jnp.where(d > 0, r, jnp.float32(0.0))


def _body(x_hbm, src_hbm, dst_hbm, out_hbm, y_hbm,
          acc_sh, deg_sh,
          src_v, dst_v, rowbuf, rowbuf2, tbuf, degbuf, ones_v, zeros_v,
          sem_a, sem_b):
    c = lax.axis_index("c")
    t = lax.axis_index("s")
    col0 = c * DH
    ebase = t * (MAIN_CH * EC)
    has_tail = t < 4
    nch = jnp.where(has_tail, MAX_CH, MAIN_CH)
    row_t = t * RPT

    def _fill_ones(i, _):
        ones_v[i, pl.ds(0, 16)] = jnp.full((16,), 1.0, jnp.float32)
        return 0
    lax.fori_loop(0, EC, _fill_ones, 0)

    def _fill_zeros(r, _):
        for cc in range(DH // 16):
            zeros_v[r, pl.ds(cc * 16, 16)] = jnp.zeros((16,), jnp.float32)
        return 0
    lax.fori_loop(0, CR, _fill_zeros, 0)

    pltpu.sync_copy(src_hbm.at[pl.ds(ebase, MAIN_CH * EC)],
                    src_v.at[pl.ds(0, MAIN_CH * EC)])
    pltpu.sync_copy(dst_hbm.at[pl.ds(t * MAIN_CH, MAIN_CH), :],
                    dst_v.at[pl.ds(0, MAIN_CH), :])

    @pl.when(has_tail)
    def _():
        pltpu.sync_copy(src_hbm.at[pl.ds(TAIL0 + t * EC, EC)],
                        src_v.at[pl.ds(MAIN_CH * EC, EC)])
        pltpu.sync_copy(dst_hbm.at[TAIL0 // EC + t], dst_v.at[MAIN_CH])

    def _zdeg(k, _):
        pltpu.sync_copy(zeros_v.at[pl.ds(0, CR), pl.ds(0, 16)],
                        deg_sh.at[pl.ds(row_t + k * CR, CR), :])
        return 0
    lax.fori_loop(0, NRC, _zdeg, 0)
    for k in range(NRC):
        pltpu.sync_copy(zeros_v, acc_sh.at[pl.ds(row_t + k * CR, CR), :])
    plsc.subcore_barrier()

    def _deg_ch(j, _):
        @pl.when(j >= 16)
        def _():
            pltpu.make_async_copy(
                ones_v, deg_sh.at[dst_v.at[0]], sem_a).wait()
        pltpu.async_copy(ones_v, deg_sh.at[dst_v.at[j]], sem_a, add=True)
        return 0
    lax.fori_loop(0, nch, _deg_ch, 0)

    def _deg_drain(r, _):
        pltpu.make_async_copy(ones_v, deg_sh.at[dst_v.at[0]], sem_a).wait()
        return 0
    lax.fori_loop(0, 16, _deg_drain, 0)
    plsc.subcore_barrier()

    pltpu.sync_copy(deg_sh.at[pl.ds(row_t, RPT), :], degbuf)

    def _dis_row(r, _):
        v = degbuf[r, pl.ds(0, 16)]
        degbuf[r, pl.ds(0, 16)] = _rsqrt16(v)
        return 0
    lax.fori_loop(0, RPT, _dis_row, 0)

    for k in range(NRC):
        row0 = row_t + k * CR

        @pl.when(row0 < N_NODES)
        def _():
            pltpu.sync_copy(x_hbm.at[pl.ds(row0, CR), pl.ds(col0, DH)], tbuf)

            def _scale_row(r, _):
                d = degbuf[k * CR + r, pl.ds(0, 16)]
                for cc in range(DH // 16):
                    sl = pl.ds(cc * 16, 16)
                    tbuf[r, sl] = tbuf[r, sl] * d
                return 0
            lax.fori_loop(0, CR, _scale_row, 0)
            pltpu.sync_copy(tbuf, y_hbm.at[c, pl.ds(row0, CR), :])
    plsc.subcore_barrier()

    for layer in range(3):
        last = layer == 2

        yref = y_hbm.at[c]

        def _gref(j):
            return yref.at[src_v.at[pl.ds(j * EC, EC)]]

        pltpu.async_copy(_gref(0), rowbuf, sem_a)

        def _pair(p, _):
            j0 = 2 * p
            pltpu.make_async_copy(_gref(j0), rowbuf, sem_a).wait()
            pltpu.async_copy(_gref(j0 + 1), rowbuf2, sem_b)
            pltpu.sync_copy(rowbuf, acc_sh.at[dst_v.at[j0]], add=True)
            pltpu.make_async_copy(_gref(j0 + 1), rowbuf2, sem_b).wait()

            @pl.when(j0 + 2 < nch)
            def _():
                pltpu.async_copy(_gref(j0 + 2), rowbuf, sem_a)
            pltpu.sync_copy(rowbuf2, acc_sh.at[dst_v.at[j0 + 1]], add=True)
            return 0
        lax.fori_loop(0, MAIN_CH // 2, _pair, 0)

        @pl.when(has_tail)
        def _():
            pltpu.make_async_copy(_gref(MAIN_CH), rowbuf, sem_a).wait()
            pltpu.sync_copy(rowbuf, acc_sh.at[dst_v.at[MAIN_CH]], add=True)
        plsc.subcore_barrier()

        for k in range(NRC):
            row0 = row_t + k * CR

            @pl.when(row0 < N_NODES)
            def _():
                pltpu.sync_copy(acc_sh.at[pl.ds(row0, CR), :], tbuf)

                def _trow(r, _):
                    d = degbuf[k * CR + r, pl.ds(0, 16)]
                    for cc in range(DH // 16):
                        sl = pl.ds(cc * 16, 16)
                        v = tbuf[r, sl]
                        v = jnp.maximum(v * d, jnp.float32(0.0))
                        if not last:
                            v = v * d
                        tbuf[r, sl] = v
                    return 0
                lax.fori_loop(0, CR, _trow, 0)
                if last:
                    pltpu.sync_copy(
                        tbuf, out_hbm.at[pl.ds(row0, CR), pl.ds(col0, DH)])
                else:
                    pltpu.sync_copy(tbuf, y_hbm.at[c, pl.ds(row0, CR), :])
                    pltpu.sync_copy(zeros_v,
                                    acc_sh.at[pl.ds(row0, CR), :])
        if not last:
            plsc.subcore_barrier()


@jax.jit
def _run(x, src, dst):
    mesh = plsc.VectorSubcoreMesh(core_axis_name="c", subcore_axis_name="s")
    f = pl.kernel(
        _body,
        out_type=(
            jax.ShapeDtypeStruct((N_NODES, D_FEAT), jnp.float32),
            jax.ShapeDtypeStruct((2, NPAD, DH), jnp.float32),
        ),
        mesh=mesh,
        compiler_params=pltpu.CompilerParams(use_tc_tiling_on_sc=False),
        scratch_types=[
            pltpu.VMEM_SHARED((NPAD, DH), jnp.float32),
            pltpu.VMEM_SHARED((NPAD, 16), jnp.float32),
            pltpu.VMEM((MAX_CH * EC,), jnp.int32),
            pltpu.VMEM((MAX_CH, EC), jnp.int32),
            pltpu.VMEM((EC, DH), jnp.float32),
            pltpu.VMEM((EC, DH), jnp.float32),
            pltpu.VMEM((CR, DH), jnp.float32),
            pltpu.VMEM((RPT, 16), jnp.float32),
            pltpu.VMEM((EC, 16), jnp.float32),
            pltpu.VMEM((CR, DH), jnp.float32),
            pltpu.SemaphoreType.DMA,
            pltpu.SemaphoreType.DMA,
        ],
    )
    out, _ = f(x, src, dst)
    return out


def kernel(x, edge_index):
    ei = edge_index.astype(jnp.int32)
    return _run(x, ei[0], ei[1].reshape(N_EDGES // EC, EC))


# speedup vs baseline: 22.7634x; 1.3270x over previous
import jax
import jax.numpy as jnp
from jax import lax
from jax.experimental import pallas as pl
from jax.experimental.pallas import tpu as pltpu
from jax.experimental.pallas import tpu_sc as plsc

N_NODES = 10000
N_EDGES = 320000
D_FEAT = 128
DH = 64
NPAD = 10240
RPT = 640
CR = 80
NRC = RPT // CR
EC = 128
MAIN_CH = 156
MAX_CH = 157
TAIL0 = MAIN_CH * EC * 16
NB = 3


def _rsqrt16(d):
    i = lax.bitcast_convert_type(d, jnp.int32)
    i = jnp.int32(0x5F3759DF) - lax.shift_right_arithmetic(i, 1)
    r = lax.bitcast_convert_type(i, jnp.float32)
    for _ in range(3):
        r = r * (jnp.float32(1.5) - jnp.float32(0.5) * d * r * r)
    return jnp.where(d > 0, r, jnp.float32(0.0))


def _body(x_hbm, src_hbm, dst_hbm, out_hbm, y_hbm,
          acc_sh, deg_sh,
          src_v, dst_v, buf0, buf1, buf2, degbuf, ones_v,
          gs0, gs1, gs2, ss0, ss1, ss2):
    c = lax.axis_index("c")
    t = lax.axis_index("s")
    col0 = c * DH
    ebase = t * (MAIN_CH * EC)
    has_tail = t < 4
    nch = jnp.where(has_tail, MAX_CH, MAIN_CH)
    row_t = t * RPT
    bufs = (buf0, buf1, buf2)
    gsems = (gs0, gs1, gs2)
    ssems = (ss0, ss1, ss2)

    def _fill_ones(i, _):
        ones_v[i, pl.ds(0, 16)] = jnp.full((16,), 1.0, jnp.float32)
        return 0
    lax.fori_loop(0, EC, _fill_ones, 0)

    def _fill_zeros(r, _):
        for cc in range(DH // 16):
            buf1[r, pl.ds(cc * 16, 16)] = jnp.zeros((16,), jnp.float32)
        return 0
    lax.fori_loop(0, CR, _fill_zeros, 0)

    pltpu.sync_copy(src_hbm.at[pl.ds(ebase, MAIN_CH * EC)],
                    src_v.at[pl.ds(0, MAIN_CH * EC)])
    pltpu.sync_copy(dst_hbm.at[pl.ds(t * MAIN_CH, MAIN_CH), :],
                    dst_v.at[pl.ds(0, MAIN_CH), :])

    @pl.when(has_tail)
    def _():
        pltpu.sync_copy(src_hbm.at[pl.ds(TAIL0 + t * EC, EC)],
                        src_v.at[pl.ds(MAIN_CH * EC, EC)])
        pltpu.sync_copy(dst_hbm.at[TAIL0 // EC + t], dst_v.at[MAIN_CH])

    for k in range(NRC):
        pltpu.sync_copy(buf1.at[pl.ds(0, CR), pl.ds(0, 16)],
                        deg_sh.at[pl.ds(row_t + k * CR, CR), :])
        pltpu.sync_copy(buf1.at[pl.ds(0, CR), :],
                        acc_sh.at[pl.ds(row_t + k * CR, CR), :])
    plsc.subcore_barrier()

    def _deg_ch(j, _):
        @pl.when(j >= 16)
        def _():
            pltpu.make_async_copy(ones_v, deg_sh.at[dst_v.at[0]], gs0).wait()
        pltpu.async_copy(ones_v, deg_sh.at[dst_v.at[j]], gs0, add=True)
        return 0
    lax.fori_loop(0, nch, _deg_ch, 0)

    def _deg_drain(r, _):
        pltpu.make_async_copy(ones_v, deg_sh.at[dst_v.at[0]], gs0).wait()
        return 0
    lax.fori_loop(0, 16, _deg_drain, 0)
    plsc.subcore_barrier()

    pltpu.sync_copy(deg_sh.at[pl.ds(row_t, RPT), :], degbuf)

    def _dis_row(r, _):
        v = degbuf[r, pl.ds(0, 16)]
        degbuf[r, pl.ds(0, 16)] = _rsqrt16(v)
        return 0
    lax.fori_loop(0, RPT, _dis_row, 0)

    for k in range(NRC):
        row0 = row_t + k * CR

        @pl.when(row0 < N_NODES)
        def _():
            pltpu.sync_copy(x_hbm.at[pl.ds(row0, CR), pl.ds(col0, DH)],
                            buf0.at[pl.ds(0, CR), :])

            def _scale_row(r, _):
                d = degbuf[k * CR + r, pl.ds(0, 16)]
                for cc in range(DH // 16):
                    sl = pl.ds(cc * 16, 16)
                    buf0[r, sl] = buf0[r, sl] * d
                return 0
            lax.fori_loop(0, CR, _scale_row, 0)
            pltpu.sync_copy(buf0.at[pl.ds(0, CR), :],
                            y_hbm.at[c, pl.ds(row0, CR), :])
    plsc.subcore_barrier()

    for layer in range(3):
        last = layer == 2
        yref = y_hbm.at[c]

        def _gref(j):
            return yref.at[src_v.at[pl.ds(j * EC, EC)]]

        for k in range(NB):
            pltpu.async_copy(_gref(k), bufs[k], gsems[k])

        def _round(q, _):
            for k in range(NB):
                j = NB * q + k
                pltpu.make_async_copy(_gref(j), bufs[k], gsems[k]).wait()
                pltpu.async_copy(bufs[k], acc_sh.at[dst_v.at[j]], ssems[k],
                                 add=True)
            for k in range(NB):
                jn = NB * q + k + NB
                pltpu.make_async_copy(
                    bufs[k], acc_sh.at[dst_v.at[0]], ssems[k]).wait()

                @pl.when(jn < nch)
                def _(k=k, jn=jn):
                    pltpu.async_copy(_gref(jn), bufs[k], gsems[k])
            return 0
        lax.fori_loop(0, MAIN_CH // NB, _round, 0)

        @pl.when(has_tail)
        def _():
            pltpu.make_async_copy(_gref(MAIN_CH), buf0, gs0).wait()
            pltpu.sync_copy(buf0, acc_sh.at[dst_v.at[MAIN_CH]], add=True)
        plsc.subcore_barrier()

        if not last:
            lax.fori_loop(0, CR, _fill_zeros, 0)
        for k in range(NRC):
            row0 = row_t + k * CR

            @pl.when(row0 < N_NODES)
            def _():
                pltpu.sync_copy(acc_sh.at[pl.ds(row0, CR), :],
                                buf0.at[pl.ds(0, CR), :])

                def _trow(r, _):
                    d = degbuf[k * CR + r, pl.ds(0, 16)]
                    for cc in range(DH // 16):
                        sl = pl.ds(cc * 16, 16)
                        v = buf0[r, sl]
                        v = jnp.maximum(v * d, jnp.float32(0.0))
                        if not last:
                            v = v * d
                        buf0[r, sl] = v
                    return 0
                lax.fori_loop(0, CR, _trow, 0)
                if last:
                    pltpu.sync_copy(
                        buf0.at[pl.ds(0, CR), :],
                        out_hbm.at[pl.ds(row0, CR), pl.ds(col0, DH)])
                else:
                    pltpu.sync_copy(buf0.at[pl.ds(0, CR), :],
                                    y_hbm.at[c, pl.ds(row0, CR), :])
                    pltpu.sync_copy(buf1.at[pl.ds(0, CR), :],
                                    acc_sh.at[pl.ds(row0, CR), :])
        if not last:
            plsc.subcore_barrier()


@jax.jit
def _run(x, src, dst):
    mesh = plsc.VectorSubcoreMesh(core_axis_name="c", subcore_axis_name="s")
    f = pl.kernel(
        _body,
        out_type=(
            jax.ShapeDtypeStruct((N_NODES, D_FEAT), jnp.float32),
            jax.ShapeDtypeStruct((2, NPAD, DH), jnp.float32),
        ),
        mesh=mesh,
        compiler_params=pltpu.CompilerParams(use_tc_tiling_on_sc=False),
        scratch_types=[
            pltpu.VMEM_SHARED((NPAD, DH), jnp.float32),
            pltpu.VMEM_SHARED((NPAD, 16), jnp.float32),
            pltpu.VMEM((MAX_CH * EC,), jnp.int32),
            pltpu.VMEM((MAX_CH, EC), jnp.int32),
            pltpu.VMEM((EC, DH), jnp.float32),
            pltpu.VMEM((EC, DH), jnp.float32),
            pltpu.VMEM((EC, DH), jnp.float32),
            pltpu.VMEM((RPT, 16), jnp.float32),
            pltpu.VMEM((EC, 16), jnp.float32),
            pltpu.SemaphoreType.DMA,
            pltpu.SemaphoreType.DMA,
            pltpu.SemaphoreType.DMA,
            pltpu.SemaphoreType.DMA,
            pltpu.SemaphoreType.DMA,
            pltpu.SemaphoreType.DMA,
        ],
    )
    out, _ = f(x, src, dst)
    return out


def kernel(x, edge_index):
    ei = edge_index.astype(jnp.int32)
    return _run(x, ei[0], ei[1].reshape(N_EDGES // EC, EC))
